# baseline (speedup 1.0000x reference)
import functools

import jax
import jax.numpy as jnp
from jax import lax
from jax.experimental import pallas as pl
from jax.experimental.pallas import tpu as pltpu

N = 10000
E = 320000
DF = 128
H = 32
K = 30

_RB = 400


def _softmax_body(x_ref, o_ref):
    x = x_ref[...]
    m = jnp.max(x, axis=-1, keepdims=True)
    e = jnp.exp(x - m)
    o_ref[...] = e / jnp.sum(e, axis=-1, keepdims=True)


def _softmax_rows(x):
    n, k = x.shape
    return pl.pallas_call(
        _softmax_body,
        grid=(n // _RB,),
        in_specs=[pl.BlockSpec((_RB, k), lambda i: (i, 0))],
        out_specs=pl.BlockSpec((_RB, k), lambda i: (i, 0)),
        out_shape=jax.ShapeDtypeStruct((n, k), jnp.float32),
    )(x)


def _corr_body(a_ref, b_ref, o_ref):
    o_ref[...] = 1.0 - lax.dot_general(
        a_ref[...], b_ref[...], (((1,), (1,)), ((), ())),
        preferred_element_type=jnp.float32)


def _corr(fx):
    n, k = fx.shape
    return pl.pallas_call(
        _corr_body,
        grid=(n // _RB,),
        in_specs=[
            pl.BlockSpec((_RB, k), lambda i: (i, 0)),
            pl.BlockSpec((n, k), lambda i: (0, 0)),
        ],
        out_specs=pl.BlockSpec((_RB, n), lambda i: (i, 0)),
        out_shape=jax.ShapeDtypeStruct((n, n), jnp.float32),
    )(fx, fx)


def kernel(X, edge_index, D, n1_pW, n1_pb, n1_sW, n1_sb, e1_pW, e1_pb,
           e1_sW, e1_sb, n2_pW, n2_pb, n2_sW, n2_sb, e2_pW, e2_pb, e2_sW,
           e2_sb, n3_pW, n3_pb, n3_sW, n3_sb):
    row = edge_index[0]
    col = edge_index[1]

    def lin(x, W, b):
        return x @ W.T + b

    def node_conv(Av, Xn, pW, pb, sW, sb, act):
        Asum = jax.ops.segment_sum(Av, row, num_segments=N)
        out = lin(Asum / D[:, None], pW, pb) + lin(Xn, sW, sb)
        return jax.nn.relu(out) if act else out

    def edge_conv(Av, Xn, pW, pb, sW, sb):
        X1 = Xn[row]
        X2 = Xn[col]
        Xd = (X1 - X2) / 2.0
        Xs = (X1 + X2) / 2.0
        Ecat = jnp.concatenate([Xd, Xs], axis=-1)
        return jax.nn.relu(lin(Ecat, pW, pb) + lin(Av, sW, sb))

    Av = jnp.abs(X[row] - X[col])
    Xn = jnp.zeros_like(X)
    Xn = node_conv(Av, Xn, n1_pW, n1_pb, n1_sW, n1_sb, True)
    Av = edge_conv(Av, Xn, e1_pW, e1_pb, e1_sW, e1_sb)
    Xn = node_conv(Av, Xn, n2_pW, n2_pb, n2_sW, n2_sb, True)
    Av = edge_conv(Av, Xn, e2_pW, e2_pb, e2_sW, e2_sb)
    FXl = node_conv(Av, Xn, n3_pW, n3_pb, n3_sW, n3_sb, False)
    FX = _softmax_rows(FXl)
    corr = _corr(FX)
    return (FX, corr)


# speedup vs baseline: 2.6775x; 2.6775x over previous
import jax
import jax.numpy as jnp
from jax import lax
from jax.experimental import pallas as pl
from jax.experimental.pallas import tpu as pltpu
from jax.experimental.pallas import tpu_sc as plsc

N = 10000
E = 320000
DF = 128
H = 32
K = 30

NC = 2
NS = 16
CH = 128
NP = 10240
ZR = NP // NS

_axes = ("core", "subcore")
_mesh_cache = []


def _mesh_():
    if not _mesh_cache:
        _mesh_cache.append(plsc.VectorSubcoreMesh(
            core_axis_name="core", subcore_axis_name="subcore"))
    return _mesh_cache[0]


def _sc_absdiff(X, row2d, col2d):
    @pl.kernel(
        out_type=jax.ShapeDtypeStruct((E, DF), jnp.float32),
        mesh=_mesh_(),
        scratch_types=[
            pltpu.VMEM((1, CH), jnp.int32),
            pltpu.VMEM((1, CH), jnp.int32),
            pltpu.VMEM((CH // 2, DF), jnp.float32),
            pltpu.VMEM((CH // 2, DF), jnp.float32),
        ],
    )
    def k(x_hbm, r_hbm, c_hbm, o_hbm, r_v, c_v, a_v, b_v):
        cid = lax.axis_index("core")
        sid = lax.axis_index("subcore")
        wid = sid * NC + cid
        GH = CH // 2

        @pl.loop(wid, E // CH, step=NC * NS)
        def _(t):
            pltpu.sync_copy(r_hbm.at[:, pl.ds(t * CH, CH)], r_v)
            pltpu.sync_copy(c_hbm.at[:, pl.ds(t * CH, CH)], c_v)
            for h in range(2):
                pltpu.sync_copy(x_hbm.at[r_v.at[0, pl.ds(h * GH, GH)]], a_v)
                pltpu.sync_copy(x_hbm.at[c_v.at[0, pl.ds(h * GH, GH)]], b_v)

                @pl.loop(0, GH)
                def _(i):
                    for j in range(0, DF, 16):
                        slc = (pl.ds(i, 1), pl.ds(j, 16))
                        a_v.at[slc][...] = jnp.abs(a_v.at[slc][...] - b_v.at[slc][...])

                pltpu.sync_copy(a_v, o_hbm.at[pl.ds(t * CH + h * GH, GH)])

    return k(X, row2d, col2d)


def _sc_gather_cat(Gcat, row2d, col2d):
    @pl.kernel(
        out_type=jax.ShapeDtypeStruct((E, H), jnp.float32),
        mesh=_mesh_(),
        scratch_types=[
            pltpu.VMEM((1, CH), jnp.int32),
            pltpu.VMEM((1, CH), jnp.int32),
            pltpu.VMEM((CH // 2, DF), jnp.float32),
            pltpu.VMEM((CH // 2, DF), jnp.float32),
            pltpu.VMEM((CH // 2, H), jnp.float32),
        ],
    )
    def k(g_hbm, r_hbm, c_hbm, o_hbm, r_v, c_v, a_v, b_v, o_v):
        cid = lax.axis_index("core")
        sid = lax.axis_index("subcore")
        wid = sid * NC + cid
        GH = CH // 2

        @pl.loop(wid, E // CH, step=NC * NS)
        def _(t):
            pltpu.sync_copy(r_hbm.at[:, pl.ds(t * CH, CH)], r_v)
            pltpu.sync_copy(c_hbm.at[:, pl.ds(t * CH, CH)], c_v)
            for h in range(2):
                pltpu.sync_copy(g_hbm.at[r_v.at[0, pl.ds(h * GH, GH)]], a_v)
                pltpu.sync_copy(g_hbm.at[c_v.at[0, pl.ds(h * GH, GH)]], b_v)

                @pl.loop(0, GH)
                def _(i):
                    for j in range(0, H, 16):
                        o_v.at[pl.ds(i, 1), pl.ds(j, 16)][...] = (
                            a_v.at[pl.ds(i, 1), pl.ds(j, 16)][...]
                            + b_v.at[pl.ds(i, 1), pl.ds(H + j, 16)][...])

                pltpu.sync_copy(o_v, o_hbm.at[pl.ds(t * CH + h * GH, GH)])

    return k(Gcat, row2d, col2d)


def _sc_segsum(P, row2d):
    @pl.kernel(
        out_type=jax.ShapeDtypeStruct((NC, NP, H), jnp.float32),
        mesh=_mesh_(),
        scratch_types=[
            pltpu.VMEM((64, H), jnp.float32),
            pltpu.VMEM_SHARED((NP, H), jnp.float32),
        ],
    )
    def k(p_hbm, r_hbm, o_hbm, z_v, acc):
        cid = lax.axis_index("core")
        sid = lax.axis_index("subcore")

        @pl.loop(0, 64)
        def _(i):
            for j in range(0, H, 16):
                z_v.at[pl.ds(i, 1), pl.ds(j, 16)][...] = jnp.zeros(
                    (1, 16), jnp.float32)

        @pl.loop(0, ZR, step=64)
        def _(zz):
            pltpu.sync_copy(z_v, acc.at[pl.ds(sid * ZR + zz, 64)])

        plsc.subcore_barrier()

        def body(p_vmem, i_vmem):
            pltpu.sync_copy(p_vmem, acc.at[i_vmem.at[0]], add=True)

        pltpu.emit_pipeline(
            body,
            grid=(E // CH,),
            in_specs=[
                pl.BlockSpec((CH, H), lambda i: (i, 0)),
                pl.BlockSpec((1, CH), lambda i: (0, i)),
            ],
            out_specs=[],
            core_axis_name=_axes,
            dimension_semantics=(pltpu.PARALLEL,),
        )(p_hbm, r_hbm)

        plsc.subcore_barrier()
        pltpu.sync_copy(acc.at[pl.ds(sid * ZR, ZR)],
                        o_hbm.at[cid, pl.ds(sid * ZR, ZR)])

    return k(P, row2d)


_BE = 2000


def _tc_edge0(Av, Wcat):
    def body(a_ref, w_ref, p_ref, s_ref):
        d = jnp.dot(a_ref[...], w_ref[...], preferred_element_type=jnp.float32)
        p_ref[...] = d[:, :H]
        s_ref[...] = d[:, H:]

    return pl.pallas_call(
        body,
        grid=(E // _BE,),
        in_specs=[
            pl.BlockSpec((_BE, DF), lambda i: (i, 0)),
            pl.BlockSpec((DF, 2 * H), lambda i: (0, 0)),
        ],
        out_specs=[
            pl.BlockSpec((_BE, H), lambda i: (i, 0)),
            pl.BlockSpec((_BE, H), lambda i: (i, 0)),
        ],
        out_shape=[jax.ShapeDtypeStruct((E, H), jnp.float32)] * 2,
    )(Av, Wcat)


def _tc_edge(Gr, S, WoutT):
    dout = WoutT.shape[1]
    two = dout == 2 * H

    def body(g_ref, s_ref, wo_ref, *outs):
        av = jnp.maximum(g_ref[...] + s_ref[...], 0.0)
        d = jnp.dot(av, wo_ref[...], preferred_element_type=jnp.float32)
        if two:
            outs[0][...] = d[:, :H]
            outs[1][...] = d[:, H:]
        else:
            outs[0][...] = d

    out_specs = [pl.BlockSpec((_BE, H), lambda i: (i, 0))]
    out_shape = [jax.ShapeDtypeStruct((E, H), jnp.float32)]
    if two:
        out_specs = out_specs * 2
        out_shape = out_shape * 2
    return pl.pallas_call(
        body,
        grid=(E // _BE,),
        in_specs=[
            pl.BlockSpec((_BE, H), lambda i: (i, 0)),
            pl.BlockSpec((_BE, H), lambda i: (i, 0)),
            pl.BlockSpec((H, dout), lambda i: (0, 0)),
        ],
        out_specs=out_specs,
        out_shape=out_shape,
    )(Gr, S, WoutT)


_NB = 1280


def _tc_node(parts, Drecip, bias, Wg1T, Wg2T, bg, Xn=None, WsT=None):
    with_s = Xn is not None

    def body(*refs):
        if with_s:
            p_ref, d_ref, b_ref, wg1_ref, wg2_ref, bg_ref, x_ref, ws_ref, \
                xo_ref, g1_ref, g2_ref = refs
        else:
            p_ref, d_ref, b_ref, wg1_ref, wg2_ref, bg_ref, \
                xo_ref, g1_ref, g2_ref = refs
        a = (p_ref[0] + p_ref[1]) * d_ref[...]
        if with_s:
            a = a + jnp.dot(x_ref[...], ws_ref[...],
                            preferred_element_type=jnp.float32)
        xn = jnp.maximum(a + b_ref[...], 0.0)
        xo_ref[...] = xn
        g1_ref[...] = jnp.dot(xn, wg1_ref[...],
                              preferred_element_type=jnp.float32) + bg_ref[...]
        g2_ref[...] = jnp.dot(xn, wg2_ref[...],
                              preferred_element_type=jnp.float32)

    in_specs = [
        pl.BlockSpec((NC, _NB, H), lambda i: (0, i, 0)),
        pl.BlockSpec((_NB, 1), lambda i: (i, 0)),
        pl.BlockSpec((1, H), lambda i: (0, 0)),
        pl.BlockSpec((H, H), lambda i: (0, 0)),
        pl.BlockSpec((H, H), lambda i: (0, 0)),
        pl.BlockSpec((1, H), lambda i: (0, 0)),
    ]
    args = [parts, Drecip, bias, Wg1T, Wg2T, bg]
    if with_s:
        in_specs += [
            pl.BlockSpec((_NB, H), lambda i: (i, 0)),
            pl.BlockSpec((H, H), lambda i: (0, 0)),
        ]
        args += [Xn, WsT]
    return pl.pallas_call(
        body,
        grid=(NP // _NB,),
        in_specs=in_specs,
        out_specs=[pl.BlockSpec((_NB, H), lambda i: (i, 0))] * 3,
        out_shape=[jax.ShapeDtypeStruct((NP, H), jnp.float32)] * 3,
    )(*args)


def _pack_g(G1, G2):
    return jnp.pad(jnp.concatenate([G1, G2], axis=1),
                   ((0, 0), (0, DF - 2 * H)))


_NB3 = 2000


def _tc_node3(parts, Drecip, Xn, WsT, bias):
    def body(p_ref, d_ref, x_ref, w_ref, b_ref, o_ref):
        a = (p_ref[0] + p_ref[1])[:, :K] * d_ref[...]
        a = a + jnp.dot(x_ref[...], w_ref[...], preferred_element_type=jnp.float32)
        a = a + b_ref[...]
        m = jnp.max(a, axis=-1, keepdims=True)
        ex = jnp.exp(a - m)
        o_ref[...] = ex / jnp.sum(ex, axis=-1, keepdims=True)

    return pl.pallas_call(
        body,
        grid=(N // _NB3,),
        in_specs=[
            pl.BlockSpec((NC, _NB3, H), lambda i: (0, i, 0)),
            pl.BlockSpec((_NB3, 1), lambda i: (i, 0)),
            pl.BlockSpec((_NB3, H), lambda i: (i, 0)),
            pl.BlockSpec((H, K), lambda i: (0, 0)),
            pl.BlockSpec((1, K), lambda i: (0, 0)),
        ],
        out_specs=pl.BlockSpec((_NB3, K), lambda i: (i, 0)),
        out_shape=jax.ShapeDtypeStruct((N, K), jnp.float32),
    )(parts, Drecip, Xn, WsT, bias)


_RB = 400


def _corr(fx):
    def body(a_ref, b_ref, o_ref):
        o_ref[...] = 1.0 - lax.dot_general(
            a_ref[...], b_ref[...], (((1,), (1,)), ((), ())),
            preferred_element_type=jnp.float32)

    n, k = fx.shape
    return pl.pallas_call(
        body,
        grid=(n // _RB,),
        in_specs=[
            pl.BlockSpec((_RB, k), lambda i: (i, 0)),
            pl.BlockSpec((n, k), lambda i: (0, 0)),
        ],
        out_specs=pl.BlockSpec((_RB, n), lambda i: (i, 0)),
        out_shape=jax.ShapeDtypeStruct((n, n), jnp.float32),
    )(fx, fx)


def kernel(X, edge_index, D, n1_pW, n1_pb, n1_sW, n1_sb, e1_pW, e1_pb,
           e1_sW, e1_sb, n2_pW, n2_pb, n2_sW, n2_sb, e2_pW, e2_pb, e2_sW,
           e2_sb, n3_pW, n3_pb, n3_sW, n3_sb):
    row2d = edge_index[0:1]
    col2d = edge_index[1:2]
    Drecip = (1.0 / D)[:, None]
    Drecip_p = jnp.pad(Drecip, ((0, NP - N), (0, 0)),
                       constant_values=1.0)

    Wcat1 = jnp.concatenate([n1_pW.T, e1_sW.T], axis=1)
    Wcat2 = jnp.concatenate([n2_pW.T, e2_sW.T], axis=1)
    W3pT = jnp.pad(n3_pW.T, ((0, 0), (0, H - K)))
    Wg1T_1 = (e1_pW[:, :H] + e1_pW[:, H:]).T * 0.5
    Wg2T_1 = (e1_pW[:, H:] - e1_pW[:, :H]).T * 0.5
    Wg1T_2 = (e2_pW[:, :H] + e2_pW[:, H:]).T * 0.5
    Wg2T_2 = (e2_pW[:, H:] - e2_pW[:, :H]).T * 0.5
    b1 = (n1_pb + n1_sb)[None, :]
    be1 = (e1_pb + e1_sb)[None, :]
    b2 = (n2_pb + n2_sb)[None, :]
    be2 = (e2_pb + e2_sb)[None, :]
    b3 = (n3_pb + n3_sb)[None, :]

    Av = _sc_absdiff(X, row2d, col2d)
    P1, S1 = _tc_edge0(Av, Wcat1)
    parts1 = _sc_segsum(P1, row2d)
    Xn1, G1a, G2a = _tc_node(parts1, Drecip_p, b1, Wg1T_1, Wg2T_1, be1)

    Gr1 = _sc_gather_cat(_pack_g(G1a, G2a), row2d, col2d)
    P2, S2 = _tc_edge(Gr1, S1, Wcat2)
    parts2 = _sc_segsum(P2, row2d)
    Xn2, G1b, G2b = _tc_node(parts2, Drecip_p, b2, Wg1T_2, Wg2T_2, be2,
                             Xn=Xn1, WsT=n2_sW.T)

    Gr2 = _sc_gather_cat(_pack_g(G1b, G2b), row2d, col2d)
    (P3,) = _tc_edge(Gr2, S2, W3pT)
    parts3 = _sc_segsum(P3, row2d)
    FX = _tc_node3(parts3, Drecip, Xn2, n3_sW.T, b3)

    corr = _corr(FX)
    return (FX, corr)


# speedup vs baseline: 3.2700x; 1.2213x over previous
import jax
import jax.numpy as jnp
from jax import lax
from jax.experimental import pallas as pl
from jax.experimental.pallas import tpu as pltpu
from jax.experimental.pallas import tpu_sc as plsc

N = 10000
E = 320000
DF = 128
H = 32
K = 30

NC = 2
NS = 16
CH = 128
NP = 10240
ZR = NP // NS

_axes = ("core", "subcore")
_mesh_cache = []


def _mesh_():
    if not _mesh_cache:
        _mesh_cache.append(plsc.VectorSubcoreMesh(
            core_axis_name="core", subcore_axis_name="subcore"))
    return _mesh_cache[0]


def _sc_absdiff(X, row2d, col2d):
    @pl.kernel(
        out_type=jax.ShapeDtypeStruct((E, DF), jnp.float32),
        mesh=_mesh_(),
        scratch_types=[
            pltpu.VMEM((1, CH), jnp.int32),
            pltpu.VMEM((1, CH), jnp.int32),
            pltpu.VMEM((CH // 2, DF), jnp.float32),
            pltpu.VMEM((CH // 2, DF), jnp.float32),
            pltpu.SemaphoreType.DMA,
            pltpu.SemaphoreType.DMA,
        ],
    )
    def k(x_hbm, r_hbm, c_hbm, o_hbm, r_v, c_v, a_v, b_v, s1, s2):
        cid = lax.axis_index("core")
        sid = lax.axis_index("subcore")
        wid = sid * NC + cid
        GH = CH // 2

        @pl.loop(wid, E // CH, step=NC * NS)
        def _(t):
            d1 = pltpu.async_copy(r_hbm.at[:, pl.ds(t * CH, CH)], r_v, s1)
            d2 = pltpu.async_copy(c_hbm.at[:, pl.ds(t * CH, CH)], c_v, s2)
            d1.wait()
            d2.wait()
            for h in range(2):
                g1 = pltpu.async_copy(
                    x_hbm.at[r_v.at[0, pl.ds(h * GH, GH)]], a_v, s1)
                g2 = pltpu.async_copy(
                    x_hbm.at[c_v.at[0, pl.ds(h * GH, GH)]], b_v, s2)
                g1.wait()
                g2.wait()

                @pl.loop(0, GH)
                def _(i):
                    for j in range(0, DF, 16):
                        slc = (pl.ds(i, 1), pl.ds(j, 16))
                        a_v.at[slc][...] = jnp.abs(a_v.at[slc][...] - b_v.at[slc][...])

                pltpu.sync_copy(a_v, o_hbm.at[pl.ds(t * CH + h * GH, GH)])

    return k(X, row2d, col2d)


def _sc_gather_cat(Gcat, row2d, col2d):
    @pl.kernel(
        out_type=jax.ShapeDtypeStruct((E, H), jnp.float32),
        mesh=_mesh_(),
        scratch_types=[
            pltpu.VMEM((1, CH), jnp.int32),
            pltpu.VMEM((1, CH), jnp.int32),
            pltpu.VMEM((CH // 2, DF), jnp.float32),
            pltpu.VMEM((CH // 2, DF), jnp.float32),
            pltpu.VMEM((CH // 2, H), jnp.float32),
            pltpu.SemaphoreType.DMA,
            pltpu.SemaphoreType.DMA,
        ],
    )
    def k(g_hbm, r_hbm, c_hbm, o_hbm, r_v, c_v, a_v, b_v, o_v, s1, s2):
        cid = lax.axis_index("core")
        sid = lax.axis_index("subcore")
        wid = sid * NC + cid
        GH = CH // 2

        @pl.loop(wid, E // CH, step=NC * NS)
        def _(t):
            d1 = pltpu.async_copy(r_hbm.at[:, pl.ds(t * CH, CH)], r_v, s1)
            d2 = pltpu.async_copy(c_hbm.at[:, pl.ds(t * CH, CH)], c_v, s2)
            d1.wait()
            d2.wait()
            for h in range(2):
                g1 = pltpu.async_copy(
                    g_hbm.at[r_v.at[0, pl.ds(h * GH, GH)]], a_v, s1)
                g2 = pltpu.async_copy(
                    g_hbm.at[c_v.at[0, pl.ds(h * GH, GH)]], b_v, s2)
                g1.wait()
                g2.wait()

                @pl.loop(0, GH)
                def _(i):
                    for j in range(0, H, 16):
                        o_v.at[pl.ds(i, 1), pl.ds(j, 16)][...] = (
                            a_v.at[pl.ds(i, 1), pl.ds(j, 16)][...]
                            + b_v.at[pl.ds(i, 1), pl.ds(H + j, 16)][...])

                pltpu.sync_copy(o_v, o_hbm.at[pl.ds(t * CH + h * GH, GH)])

    return k(Gcat, row2d, col2d)


def _sc_segsum(P, row2d):
    @pl.kernel(
        out_type=jax.ShapeDtypeStruct((NC, NP, H), jnp.float32),
        mesh=_mesh_(),
        scratch_types=[
            pltpu.VMEM((64, H), jnp.float32),
            pltpu.VMEM_SHARED((NP, H), jnp.float32),
        ],
    )
    def k(p_hbm, r_hbm, o_hbm, z_v, acc):
        cid = lax.axis_index("core")
        sid = lax.axis_index("subcore")

        @pl.loop(0, 64)
        def _(i):
            for j in range(0, H, 16):
                z_v.at[pl.ds(i, 1), pl.ds(j, 16)][...] = jnp.zeros(
                    (1, 16), jnp.float32)

        @pl.loop(0, ZR, step=64)
        def _(zz):
            pltpu.sync_copy(z_v, acc.at[pl.ds(sid * ZR + zz, 64)])

        plsc.subcore_barrier()

        def body(p_vmem, i_vmem):
            pltpu.sync_copy(p_vmem, acc.at[i_vmem.at[0]], add=True)

        pltpu.emit_pipeline(
            body,
            grid=(E // CH,),
            in_specs=[
                pl.BlockSpec((CH, H), lambda i: (i, 0)),
                pl.BlockSpec((1, CH), lambda i: (0, i)),
            ],
            out_specs=[],
            core_axis_name=_axes,
            dimension_semantics=(pltpu.PARALLEL,),
        )(p_hbm, r_hbm)

        plsc.subcore_barrier()
        pltpu.sync_copy(acc.at[pl.ds(sid * ZR, ZR)],
                        o_hbm.at[cid, pl.ds(sid * ZR, ZR)])

    return k(P, row2d)


_BE = 2000


def _tc_edge0(Av, Wcat):
    def body(a_ref, w_ref, p_ref, s_ref):
        d = jnp.dot(a_ref[...], w_ref[...], preferred_element_type=jnp.float32)
        p_ref[...] = d[:, :H]
        s_ref[...] = d[:, H:]

    return pl.pallas_call(
        body,
        grid=(E // _BE,),
        in_specs=[
            pl.BlockSpec((_BE, DF), lambda i: (i, 0)),
            pl.BlockSpec((DF, 2 * H), lambda i: (0, 0)),
        ],
        out_specs=[
            pl.BlockSpec((_BE, H), lambda i: (i, 0)),
            pl.BlockSpec((_BE, H), lambda i: (i, 0)),
        ],
        out_shape=[jax.ShapeDtypeStruct((E, H), jnp.float32)] * 2,
    )(Av, Wcat)


def _tc_edge(Gr, S, WoutT):
    dout = WoutT.shape[1]
    two = dout == 2 * H

    def body(g_ref, s_ref, wo_ref, *outs):
        av = jnp.maximum(g_ref[...] + s_ref[...], 0.0)
        d = jnp.dot(av, wo_ref[...], preferred_element_type=jnp.float32)
        if two:
            outs[0][...] = d[:, :H]
            outs[1][...] = d[:, H:]
        else:
            outs[0][...] = d

    out_specs = [pl.BlockSpec((_BE, H), lambda i: (i, 0))]
    out_shape = [jax.ShapeDtypeStruct((E, H), jnp.float32)]
    if two:
        out_specs = out_specs * 2
        out_shape = out_shape * 2
    return pl.pallas_call(
        body,
        grid=(E // _BE,),
        in_specs=[
            pl.BlockSpec((_BE, H), lambda i: (i, 0)),
            pl.BlockSpec((_BE, H), lambda i: (i, 0)),
            pl.BlockSpec((H, dout), lambda i: (0, 0)),
        ],
        out_specs=out_specs,
        out_shape=out_shape,
    )(Gr, S, WoutT)


_NB = 1280


def _tc_node(parts, Drecip, bias, Wg1T, Wg2T, bg, Xn=None, WsT=None):
    with_s = Xn is not None

    def body(*refs):
        if with_s:
            p_ref, d_ref, b_ref, wg1_ref, wg2_ref, bg_ref, x_ref, ws_ref, \
                xo_ref, g1_ref, g2_ref = refs
        else:
            p_ref, d_ref, b_ref, wg1_ref, wg2_ref, bg_ref, \
                xo_ref, g1_ref, g2_ref = refs
        a = (p_ref[0] + p_ref[1]) * d_ref[...]
        if with_s:
            a = a + jnp.dot(x_ref[...], ws_ref[...],
                            preferred_element_type=jnp.float32)
        xn = jnp.maximum(a + b_ref[...], 0.0)
        xo_ref[...] = xn
        g1_ref[...] = jnp.dot(xn, wg1_ref[...],
                              preferred_element_type=jnp.float32) + bg_ref[...]
        g2_ref[...] = jnp.dot(xn, wg2_ref[...],
                              preferred_element_type=jnp.float32)

    in_specs = [
        pl.BlockSpec((NC, _NB, H), lambda i: (0, i, 0)),
        pl.BlockSpec((_NB, 1), lambda i: (i, 0)),
        pl.BlockSpec((1, H), lambda i: (0, 0)),
        pl.BlockSpec((H, H), lambda i: (0, 0)),
        pl.BlockSpec((H, H), lambda i: (0, 0)),
        pl.BlockSpec((1, H), lambda i: (0, 0)),
    ]
    args = [parts, Drecip, bias, Wg1T, Wg2T, bg]
    if with_s:
        in_specs += [
            pl.BlockSpec((_NB, H), lambda i: (i, 0)),
            pl.BlockSpec((H, H), lambda i: (0, 0)),
        ]
        args += [Xn, WsT]
    return pl.pallas_call(
        body,
        grid=(NP // _NB,),
        in_specs=in_specs,
        out_specs=[pl.BlockSpec((_NB, H), lambda i: (i, 0))] * 3,
        out_shape=[jax.ShapeDtypeStruct((NP, H), jnp.float32)] * 3,
    )(*args)


def _pack_g(G1, G2):
    return jnp.pad(jnp.concatenate([G1, G2], axis=1),
                   ((0, 0), (0, DF - 2 * H)))


_NB3 = 2000


def _tc_node3(parts, Drecip, Xn, WsT, bias):
    def body(p_ref, d_ref, x_ref, w_ref, b_ref, o_ref):
        a = (p_ref[0] + p_ref[1])[:, :K] * d_ref[...]
        a = a + jnp.dot(x_ref[...], w_ref[...], preferred_element_type=jnp.float32)
        a = a + b_ref[...]
        m = jnp.max(a, axis=-1, keepdims=True)
        ex = jnp.exp(a - m)
        o_ref[...] = ex / jnp.sum(ex, axis=-1, keepdims=True)

    return pl.pallas_call(
        body,
        grid=(N // _NB3,),
        in_specs=[
            pl.BlockSpec((NC, _NB3, H), lambda i: (0, i, 0)),
            pl.BlockSpec((_NB3, 1), lambda i: (i, 0)),
            pl.BlockSpec((_NB3, H), lambda i: (i, 0)),
            pl.BlockSpec((H, K), lambda i: (0, 0)),
            pl.BlockSpec((1, K), lambda i: (0, 0)),
        ],
        out_specs=pl.BlockSpec((_NB3, K), lambda i: (i, 0)),
        out_shape=jax.ShapeDtypeStruct((N, K), jnp.float32),
    )(parts, Drecip, Xn, WsT, bias)


_RB = 400


def _corr(fx):
    def body(a_ref, b_ref, o_ref):
        o_ref[...] = 1.0 - lax.dot_general(
            a_ref[...], b_ref[...], (((1,), (1,)), ((), ())),
            preferred_element_type=jnp.float32)

    n, k = fx.shape
    return pl.pallas_call(
        body,
        grid=(n // _RB,),
        in_specs=[
            pl.BlockSpec((_RB, k), lambda i: (i, 0)),
            pl.BlockSpec((n, k), lambda i: (0, 0)),
        ],
        out_specs=pl.BlockSpec((_RB, n), lambda i: (i, 0)),
        out_shape=jax.ShapeDtypeStruct((n, n), jnp.float32),
    )(fx, fx)


def kernel(X, edge_index, D, n1_pW, n1_pb, n1_sW, n1_sb, e1_pW, e1_pb,
           e1_sW, e1_sb, n2_pW, n2_pb, n2_sW, n2_sb, e2_pW, e2_pb, e2_sW,
           e2_sb, n3_pW, n3_pb, n3_sW, n3_sb):
    row2d = edge_index[0:1]
    col2d = edge_index[1:2]
    Drecip = (1.0 / D)[:, None]
    Drecip_p = jnp.pad(Drecip, ((0, NP - N), (0, 0)),
                       constant_values=1.0)

    Wcat1 = jnp.concatenate([n1_pW.T, e1_sW.T], axis=1)
    Wcat2 = jnp.concatenate([n2_pW.T, e2_sW.T], axis=1)
    W3pT = jnp.pad(n3_pW.T, ((0, 0), (0, H - K)))
    Wg1T_1 = (e1_pW[:, :H] + e1_pW[:, H:]).T * 0.5
    Wg2T_1 = (e1_pW[:, H:] - e1_pW[:, :H]).T * 0.5
    Wg1T_2 = (e2_pW[:, :H] + e2_pW[:, H:]).T * 0.5
    Wg2T_2 = (e2_pW[:, H:] - e2_pW[:, :H]).T * 0.5
    b1 = (n1_pb + n1_sb)[None, :]
    be1 = (e1_pb + e1_sb)[None, :]
    b2 = (n2_pb + n2_sb)[None, :]
    be2 = (e2_pb + e2_sb)[None, :]
    b3 = (n3_pb + n3_sb)[None, :]

    Av = _sc_absdiff(X, row2d, col2d)
    P1, S1 = _tc_edge0(Av, Wcat1)
    parts1 = _sc_segsum(P1, row2d)
    Xn1, G1a, G2a = _tc_node(parts1, Drecip_p, b1, Wg1T_1, Wg2T_1, be1)

    Gr1 = _sc_gather_cat(_pack_g(G1a, G2a), row2d, col2d)
    P2, S2 = _tc_edge(Gr1, S1, Wcat2)
    parts2 = _sc_segsum(P2, row2d)
    Xn2, G1b, G2b = _tc_node(parts2, Drecip_p, b2, Wg1T_2, Wg2T_2, be2,
                             Xn=Xn1, WsT=n2_sW.T)

    Gr2 = _sc_gather_cat(_pack_g(G1b, G2b), row2d, col2d)
    (P3,) = _tc_edge(Gr2, S2, W3pT)
    parts3 = _sc_segsum(P3, row2d)
    FX = _tc_node3(parts3, Drecip, Xn2, n3_sW.T, b3)

    corr = _corr(FX)
    return (FX, corr)


# speedup vs baseline: 3.5005x; 1.0705x over previous
import jax
import jax.numpy as jnp
from jax import lax
from jax.experimental import pallas as pl
from jax.experimental.pallas import tpu as pltpu
from jax.experimental.pallas import tpu_sc as plsc

N = 10000
E = 320000
DF = 128
H = 32
K = 30

NC = 2
NS = 16
CH = 128
NP = 10240
ZR = NP // NS

_axes = ("core", "subcore")
_mesh_cache = []


def _mesh_():
    if not _mesh_cache:
        _mesh_cache.append(plsc.VectorSubcoreMesh(
            core_axis_name="core", subcore_axis_name="subcore"))
    return _mesh_cache[0]


def _sc_absdiff(X, row2d, col2d):
    @pl.kernel(
        out_type=jax.ShapeDtypeStruct((E, DF), jnp.float32),
        mesh=_mesh_(),
        scratch_types=[
            pltpu.VMEM((1, CH), jnp.int32),
            pltpu.VMEM((1, CH), jnp.int32),
            pltpu.VMEM((CH // 2, DF), jnp.float32),
            pltpu.VMEM((CH // 2, DF), jnp.float32),
            pltpu.VMEM((CH // 2, DF), jnp.float32),
            pltpu.VMEM((CH // 2, DF), jnp.float32),
            pltpu.SemaphoreType.DMA,
            pltpu.SemaphoreType.DMA,
            pltpu.SemaphoreType.DMA,
            pltpu.SemaphoreType.DMA,
        ],
    )
    def k(x_hbm, r_hbm, c_hbm, o_hbm, r_v, c_v, a0, b0, a1, b1,
          s1, s2, s3, s4):
        cid = lax.axis_index("core")
        sid = lax.axis_index("subcore")
        wid = sid * NC + cid
        GH = CH // 2

        @pl.loop(wid, E // CH, step=NC * NS)
        def _(t):
            d1 = pltpu.async_copy(r_hbm.at[:, pl.ds(t * CH, CH)], r_v, s1)
            d2 = pltpu.async_copy(c_hbm.at[:, pl.ds(t * CH, CH)], c_v, s2)
            d1.wait()
            d2.wait()
            bufs = ((a0, b0, s1, s2), (a1, b1, s3, s4))
            pend = []
            for h in range(2):
                a_v, b_v, sa, sb = bufs[h]
                pend.append((
                    pltpu.async_copy(
                        x_hbm.at[r_v.at[0, pl.ds(h * GH, GH)]], a_v, sa),
                    pltpu.async_copy(
                        x_hbm.at[c_v.at[0, pl.ds(h * GH, GH)]], b_v, sb)))
            for h in range(2):
                a_v, b_v, sa, sb = bufs[h]
                pend[h][0].wait()
                pend[h][1].wait()

                @pl.loop(0, GH)
                def _(i):
                    for j in range(0, DF, 16):
                        slc = (pl.ds(i, 1), pl.ds(j, 16))
                        a_v.at[slc][...] = jnp.abs(a_v.at[slc][...] - b_v.at[slc][...])

                pltpu.sync_copy(a_v, o_hbm.at[pl.ds(t * CH + h * GH, GH)])

    return k(X, row2d, col2d)


def _sc_gather_cat(Gcat, row2d, col2d):
    @pl.kernel(
        out_type=jax.ShapeDtypeStruct((E, H), jnp.float32),
        mesh=_mesh_(),
        scratch_types=[
            pltpu.VMEM((1, CH), jnp.int32),
            pltpu.VMEM((1, CH), jnp.int32),
            pltpu.VMEM((CH // 2, DF), jnp.float32),
            pltpu.VMEM((CH // 2, DF), jnp.float32),
            pltpu.VMEM((CH // 2, DF), jnp.float32),
            pltpu.VMEM((CH // 2, DF), jnp.float32),
            pltpu.VMEM((CH // 2, H), jnp.float32),
            pltpu.SemaphoreType.DMA,
            pltpu.SemaphoreType.DMA,
            pltpu.SemaphoreType.DMA,
            pltpu.SemaphoreType.DMA,
        ],
    )
    def k(g_hbm, r_hbm, c_hbm, o_hbm, r_v, c_v, a0, b0, a1, b1, o_v,
          s1, s2, s3, s4):
        cid = lax.axis_index("core")
        sid = lax.axis_index("subcore")
        wid = sid * NC + cid
        GH = CH // 2

        @pl.loop(wid, E // CH, step=NC * NS)
        def _(t):
            d1 = pltpu.async_copy(r_hbm.at[:, pl.ds(t * CH, CH)], r_v, s1)
            d2 = pltpu.async_copy(c_hbm.at[:, pl.ds(t * CH, CH)], c_v, s2)
            d1.wait()
            d2.wait()
            bufs = ((a0, b0, s1, s2), (a1, b1, s3, s4))
            pend = []
            for h in range(2):
                a_v, b_v, sa, sb = bufs[h]
                pend.append((
                    pltpu.async_copy(
                        g_hbm.at[r_v.at[0, pl.ds(h * GH, GH)]], a_v, sa),
                    pltpu.async_copy(
                        g_hbm.at[c_v.at[0, pl.ds(h * GH, GH)]], b_v, sb)))
            for h in range(2):
                a_v, b_v, sa, sb = bufs[h]
                pend[h][0].wait()
                pend[h][1].wait()

                @pl.loop(0, GH)
                def _(i):
                    for j in range(0, H, 16):
                        o_v.at[pl.ds(i, 1), pl.ds(j, 16)][...] = (
                            a_v.at[pl.ds(i, 1), pl.ds(j, 16)][...]
                            + b_v.at[pl.ds(i, 1), pl.ds(H + j, 16)][...])

                pltpu.sync_copy(o_v, o_hbm.at[pl.ds(t * CH + h * GH, GH)])

    return k(Gcat, row2d, col2d)


def _sc_segsum(P, row2d):
    @pl.kernel(
        out_type=jax.ShapeDtypeStruct((NC, NP, H), jnp.float32),
        mesh=_mesh_(),
        scratch_types=[
            pltpu.VMEM((64, H), jnp.float32),
            pltpu.VMEM_SHARED((NP, H), jnp.float32),
        ],
    )
    def k(p_hbm, r_hbm, o_hbm, z_v, acc):
        cid = lax.axis_index("core")
        sid = lax.axis_index("subcore")

        @pl.loop(0, 64)
        def _(i):
            for j in range(0, H, 16):
                z_v.at[pl.ds(i, 1), pl.ds(j, 16)][...] = jnp.zeros(
                    (1, 16), jnp.float32)

        @pl.loop(0, ZR, step=64)
        def _(zz):
            pltpu.sync_copy(z_v, acc.at[pl.ds(sid * ZR + zz, 64)])

        plsc.subcore_barrier()

        def body(p_vmem, i_vmem):
            pltpu.sync_copy(p_vmem, acc.at[i_vmem.at[0]], add=True)

        pltpu.emit_pipeline(
            body,
            grid=(E // CH,),
            in_specs=[
                pl.BlockSpec((CH, H), lambda i: (i, 0)),
                pl.BlockSpec((1, CH), lambda i: (0, i)),
            ],
            out_specs=[],
            core_axis_name=_axes,
            dimension_semantics=(pltpu.PARALLEL,),
        )(p_hbm, r_hbm)

        plsc.subcore_barrier()
        pltpu.sync_copy(acc.at[pl.ds(sid * ZR, ZR)],
                        o_hbm.at[cid, pl.ds(sid * ZR, ZR)])

    return k(P, row2d)


_BE = 2000


def _tc_edge0(Av, Wcat):
    def body(a_ref, w_ref, p_ref, s_ref):
        d = jnp.dot(a_ref[...], w_ref[...], preferred_element_type=jnp.float32)
        p_ref[...] = d[:, :H]
        s_ref[...] = d[:, H:]

    return pl.pallas_call(
        body,
        grid=(E // _BE,),
        in_specs=[
            pl.BlockSpec((_BE, DF), lambda i: (i, 0)),
            pl.BlockSpec((DF, 2 * H), lambda i: (0, 0)),
        ],
        out_specs=[
            pl.BlockSpec((_BE, H), lambda i: (i, 0)),
            pl.BlockSpec((_BE, H), lambda i: (i, 0)),
        ],
        out_shape=[jax.ShapeDtypeStruct((E, H), jnp.float32)] * 2,
    )(Av, Wcat)


def _tc_edge(Gr, S, WoutT):
    dout = WoutT.shape[1]
    two = dout == 2 * H

    def body(g_ref, s_ref, wo_ref, *outs):
        av = jnp.maximum(g_ref[...] + s_ref[...], 0.0)
        d = jnp.dot(av, wo_ref[...], preferred_element_type=jnp.float32)
        if two:
            outs[0][...] = d[:, :H]
            outs[1][...] = d[:, H:]
        else:
            outs[0][...] = d

    out_specs = [pl.BlockSpec((_BE, H), lambda i: (i, 0))]
    out_shape = [jax.ShapeDtypeStruct((E, H), jnp.float32)]
    if two:
        out_specs = out_specs * 2
        out_shape = out_shape * 2
    return pl.pallas_call(
        body,
        grid=(E // _BE,),
        in_specs=[
            pl.BlockSpec((_BE, H), lambda i: (i, 0)),
            pl.BlockSpec((_BE, H), lambda i: (i, 0)),
            pl.BlockSpec((H, dout), lambda i: (0, 0)),
        ],
        out_specs=out_specs,
        out_shape=out_shape,
    )(Gr, S, WoutT)


_NB = 1280


def _tc_node(parts, Drecip, bias, Wg1T, Wg2T, bg, Xn=None, WsT=None):
    with_s = Xn is not None

    def body(*refs):
        if with_s:
            p_ref, d_ref, b_ref, wg1_ref, wg2_ref, bg_ref, x_ref, ws_ref, \
                xo_ref, g1_ref, g2_ref = refs
        else:
            p_ref, d_ref, b_ref, wg1_ref, wg2_ref, bg_ref, \
                xo_ref, g1_ref, g2_ref = refs
        a = (p_ref[0] + p_ref[1]) * d_ref[...]
        if with_s:
            a = a + jnp.dot(x_ref[...], ws_ref[...],
                            preferred_element_type=jnp.float32)
        xn = jnp.maximum(a + b_ref[...], 0.0)
        xo_ref[...] = xn
        g1_ref[...] = jnp.dot(xn, wg1_ref[...],
                              preferred_element_type=jnp.float32) + bg_ref[...]
        g2_ref[...] = jnp.dot(xn, wg2_ref[...],
                              preferred_element_type=jnp.float32)

    in_specs = [
        pl.BlockSpec((NC, _NB, H), lambda i: (0, i, 0)),
        pl.BlockSpec((_NB, 1), lambda i: (i, 0)),
        pl.BlockSpec((1, H), lambda i: (0, 0)),
        pl.BlockSpec((H, H), lambda i: (0, 0)),
        pl.BlockSpec((H, H), lambda i: (0, 0)),
        pl.BlockSpec((1, H), lambda i: (0, 0)),
    ]
    args = [parts, Drecip, bias, Wg1T, Wg2T, bg]
    if with_s:
        in_specs += [
            pl.BlockSpec((_NB, H), lambda i: (i, 0)),
            pl.BlockSpec((H, H), lambda i: (0, 0)),
        ]
        args += [Xn, WsT]
    return pl.pallas_call(
        body,
        grid=(NP // _NB,),
        in_specs=in_specs,
        out_specs=[pl.BlockSpec((_NB, H), lambda i: (i, 0))] * 3,
        out_shape=[jax.ShapeDtypeStruct((NP, H), jnp.float32)] * 3,
    )(*args)


def _pack_g(G1, G2):
    return jnp.pad(jnp.concatenate([G1, G2], axis=1),
                   ((0, 0), (0, DF - 2 * H)))


_NB3 = 2000


def _tc_node3(parts, Drecip, Xn, WsT, bias):
    def body(p_ref, d_ref, x_ref, w_ref, b_ref, o_ref):
        a = (p_ref[0] + p_ref[1])[:, :K] * d_ref[...]
        a = a + jnp.dot(x_ref[...], w_ref[...], preferred_element_type=jnp.float32)
        a = a + b_ref[...]
        m = jnp.max(a, axis=-1, keepdims=True)
        ex = jnp.exp(a - m)
        o_ref[...] = ex / jnp.sum(ex, axis=-1, keepdims=True)

    return pl.pallas_call(
        body,
        grid=(N // _NB3,),
        in_specs=[
            pl.BlockSpec((NC, _NB3, H), lambda i: (0, i, 0)),
            pl.BlockSpec((_NB3, 1), lambda i: (i, 0)),
            pl.BlockSpec((_NB3, H), lambda i: (i, 0)),
            pl.BlockSpec((H, K), lambda i: (0, 0)),
            pl.BlockSpec((1, K), lambda i: (0, 0)),
        ],
        out_specs=pl.BlockSpec((_NB3, K), lambda i: (i, 0)),
        out_shape=jax.ShapeDtypeStruct((N, K), jnp.float32),
    )(parts, Drecip, Xn, WsT, bias)


_RB = 400


def _corr(fx):
    def body(a_ref, b_ref, o_ref):
        o_ref[...] = 1.0 - lax.dot_general(
            a_ref[...], b_ref[...], (((1,), (1,)), ((), ())),
            preferred_element_type=jnp.float32)

    n, k = fx.shape
    return pl.pallas_call(
        body,
        grid=(n // _RB,),
        in_specs=[
            pl.BlockSpec((_RB, k), lambda i: (i, 0)),
            pl.BlockSpec((n, k), lambda i: (0, 0)),
        ],
        out_specs=pl.BlockSpec((_RB, n), lambda i: (i, 0)),
        out_shape=jax.ShapeDtypeStruct((n, n), jnp.float32),
    )(fx, fx)


def kernel(X, edge_index, D, n1_pW, n1_pb, n1_sW, n1_sb, e1_pW, e1_pb,
           e1_sW, e1_sb, n2_pW, n2_pb, n2_sW, n2_sb, e2_pW, e2_pb, e2_sW,
           e2_sb, n3_pW, n3_pb, n3_sW, n3_sb):
    row2d = edge_index[0:1]
    col2d = edge_index[1:2]
    Drecip = (1.0 / D)[:, None]
    Drecip_p = jnp.pad(Drecip, ((0, NP - N), (0, 0)),
                       constant_values=1.0)

    Wcat1 = jnp.concatenate([n1_pW.T, e1_sW.T], axis=1)
    Wcat2 = jnp.concatenate([n2_pW.T, e2_sW.T], axis=1)
    W3pT = jnp.pad(n3_pW.T, ((0, 0), (0, H - K)))
    Wg1T_1 = (e1_pW[:, :H] + e1_pW[:, H:]).T * 0.5
    Wg2T_1 = (e1_pW[:, H:] - e1_pW[:, :H]).T * 0.5
    Wg1T_2 = (e2_pW[:, :H] + e2_pW[:, H:]).T * 0.5
    Wg2T_2 = (e2_pW[:, H:] - e2_pW[:, :H]).T * 0.5
    b1 = (n1_pb + n1_sb)[None, :]
    be1 = (e1_pb + e1_sb)[None, :]
    b2 = (n2_pb + n2_sb)[None, :]
    be2 = (e2_pb + e2_sb)[None, :]
    b3 = (n3_pb + n3_sb)[None, :]

    Av = _sc_absdiff(X, row2d, col2d)
    P1, S1 = _tc_edge0(Av, Wcat1)
    parts1 = _sc_segsum(P1, row2d)
    Xn1, G1a, G2a = _tc_node(parts1, Drecip_p, b1, Wg1T_1, Wg2T_1, be1)

    Gr1 = _sc_gather_cat(_pack_g(G1a, G2a), row2d, col2d)
    P2, S2 = _tc_edge(Gr1, S1, Wcat2)
    parts2 = _sc_segsum(P2, row2d)
    Xn2, G1b, G2b = _tc_node(parts2, Drecip_p, b2, Wg1T_2, Wg2T_2, be2,
                             Xn=Xn1, WsT=n2_sW.T)

    Gr2 = _sc_gather_cat(_pack_g(G1b, G2b), row2d, col2d)
    (P3,) = _tc_edge(Gr2, S2, W3pT)
    parts3 = _sc_segsum(P3, row2d)
    FX = _tc_node3(parts3, Drecip, Xn2, n3_sW.T, b3)

    corr = _corr(FX)
    return (FX, corr)
